# Initial kernel scaffold; baseline (speedup 1.0000x reference)
import jax
import jax.numpy as jnp
from jax.experimental import pallas as pl


def kernel(inputs, Wg, W1, W2, W3):
    raise NotImplementedError("write your pallas kernel here")


# speedup vs baseline: 1.1330x; 1.1330x over previous
import jax
import jax.numpy as jnp
from jax.experimental import pallas as pl
from jax.experimental.pallas import tpu as pltpu

EMBED = 768
INTER = 1024
NEXP = 64


def _moe_body(x_ref, wg_ref, w1_ref, w2_ref, w3_ref, out_ref, ew_ref, xb_ref):
    e = pl.program_id(0)

    @pl.when(e == 0)
    def _gate():
        x = x_ref[...]
        g = jnp.dot(x, wg_ref[...], preferred_element_type=jnp.float32)
        idx = jax.lax.broadcasted_iota(jnp.int32, g.shape, 1)
        m1 = jnp.max(g, axis=1, keepdims=True)
        a1 = jnp.min(jnp.where(g == m1, idx, NEXP), axis=1, keepdims=True)
        g2 = jnp.where(idx == a1, -jnp.inf, g)
        m2 = jnp.max(g2, axis=1, keepdims=True)
        a2 = jnp.min(jnp.where(g2 == m2, idx, NEXP), axis=1, keepdims=True)
        w_top = 1.0 / (1.0 + jnp.exp(m2 - m1))
        ew = jnp.where(idx == a1, w_top, 0.0) + jnp.where(idx == a2, 1.0 - w_top, 0.0)
        ew_ref[...] = ew
        xb_ref[...] = x.astype(jnp.bfloat16)
        out_ref[...] = jnp.zeros_like(out_ref)

    xb = xb_ref[...]
    w1 = w1_ref[0].astype(jnp.bfloat16)
    w3 = w3_ref[0].astype(jnp.bfloat16)
    w2 = w2_ref[0].astype(jnp.bfloat16)
    h1 = jnp.dot(xb, w1, preferred_element_type=jnp.float32)
    h3 = jnp.dot(xb, w3, preferred_element_type=jnp.float32)
    h = (h1 * jax.nn.sigmoid(h1)) * h3
    o = jnp.dot(h.astype(jnp.bfloat16), w2, preferred_element_type=jnp.float32)
    idx = jax.lax.broadcasted_iota(jnp.int32, ew_ref.shape, 1)
    col = jnp.sum(jnp.where(idx == e, ew_ref[...], 0.0), axis=1, keepdims=True)
    out_ref[...] += o * col


def kernel(inputs, Wg, W1, W2, W3):
    B, S, D = inputs.shape
    T = B * S
    x = inputs.reshape(T, D)

    out = pl.pallas_call(
        _moe_body,
        grid=(NEXP,),
        in_specs=[
            pl.BlockSpec((T, D), lambda e: (0, 0)),
            pl.BlockSpec((D, NEXP), lambda e: (0, 0)),
            pl.BlockSpec((1, D, INTER), lambda e: (e, 0, 0)),
            pl.BlockSpec((1, INTER, D), lambda e: (e, 0, 0)),
            pl.BlockSpec((1, D, INTER), lambda e: (e, 0, 0)),
        ],
        out_specs=pl.BlockSpec((T, D), lambda e: (0, 0)),
        out_shape=jax.ShapeDtypeStruct((T, D), jnp.float32),
        scratch_shapes=[
            pltpu.VMEM((T, NEXP), jnp.float32),
            pltpu.VMEM((T, D), jnp.bfloat16),
        ],
        compiler_params=pltpu.CompilerParams(
            dimension_semantics=("arbitrary",),
        ),
    )(x, Wg, W1, W2, W3)
    return out.reshape(B, S, D)
